# baseline (speedup 1.0000x reference)
import functools

import jax
import jax.numpy as jnp
from jax import lax
from jax.experimental import pallas as pl
from jax.experimental.pallas import tpu as pltpu
from jax.experimental.pallas import tpu_sc as plsc

DIM = 64
PDIM = 128
RVQ = 4
NC = 2
NS = 16
NW = NC * NS
FULL_TILES = 6
TAIL_W = 32
SUB_ROWS = 2
SPT = 8 // SUB_ROWS


def _pipeline(idx_copy, out_copy, fire_gathers, wait_gathers, compute,
              n_tiles):
    idx_copy(0, 0, sync=True)
    fire_gathers(0, 0, 0)
    if n_tiles > 1:
        idx_copy(1, 1).start()

    def pair(j, carry):
        for tp in range(2):
            tau = 2 * j + tp
            for u in range(SPT):
                bp = u % 2
                nb = 1 - bp
                if u < SPT - 1:
                    fire_gathers((u + 1) * SUB_ROWS, tp, nb)
                else:
                    @pl.when(tau + 1 < n_tiles)
                    def _():
                        idx_copy(tau + 1, 1 - tp).wait()
                        fire_gathers(0, 1 - tp, nb)
                wait_gathers(u * SUB_ROWS, tp, bp)
                if u == SPT - 1:
                    @pl.when(tau + 2 < n_tiles)
                    def _():
                        idx_copy(tau + 2, tp).start()
                if u >= 2:
                    for cp in out_copy(tau, u - 2, bp):
                        cp.wait()
                else:
                    @pl.when(tau >= 1)
                    def _():
                        for cp in out_copy(tau - 1, u + 2, bp):
                            cp.wait()
                compute(bp)
                for cp in out_copy(tau, u, bp):
                    cp.start()
        return carry

    lax.fori_loop(0, n_tiles // 2, pair, 0)
    for cp in out_copy(n_tiles - 1, SPT - 2, 0):
        cp.wait()
    for cp in out_copy(n_tiles - 1, SPT - 1, 1):
        cp.wait()


def _build(B, L):
    blocks = B // 8
    t1_tiles = blocks * FULL_TILES
    assert t1_tiles % NW == 0
    tiles_pw = t1_tiles // NW
    assert tiles_pw % 2 == 0
    tail_rows = B * TAIL_W // 128
    assert tail_rows % (NW * 8) == 0
    t2_tiles = tail_rows // (NW * 8)
    assert t2_tiles % 2 == 0
    mesh = plsc.VectorSubcoreMesh(core_axis_name="c", subcore_axis_name="s")

    @functools.partial(
        pl.kernel,
        out_type=jax.ShapeDtypeStruct((B, L, DIM), jnp.float32),
        mesh=mesh,
        scratch_types=[
            pltpu.VMEM((2, 8, 128), jnp.int32),
            pltpu.VMEM((2, SUB_ROWS * 128, PDIM), jnp.float32),
            pltpu.VMEM((2, SUB_ROWS * 32, DIM), jnp.float32),
            pltpu.SemaphoreType.DMA,
            pltpu.SemaphoreType.DMA,
            pltpu.SemaphoreType.DMA,
            pltpu.SemaphoreType.DMA,
            pltpu.SemaphoreType.DMA,
            pltpu.SemaphoreType.DMA,
        ],
        compiler_params=pltpu.CompilerParams(use_tc_tiling_on_sc=True),
    )
    def k(x_hbm, xt_hbm, emb_hbm, out_hbm, idx_v, g_v, out_v,
          si0, si1, sg0, sg1, so0, so1):
        sem_i, sem_g, sem_o = (si0, si1), (sg0, sg1), (so0, so1)
        wid = lax.axis_index("s") * NC + lax.axis_index("c")

        def fire_gathers(row0, slot, bp):
            for r in range(SUB_ROWS):
                pltpu.make_async_copy(
                    emb_hbm.at[idx_v.at[slot, row0 + r]],
                    g_v.at[bp, pl.ds(r * 128, 128)],
                    sem_g[bp],
                ).start()

        def wait_gathers(row0, slot, bp):
            for r in range(SUB_ROWS):
                pltpu.make_async_copy(
                    emb_hbm.at[idx_v.at[slot, row0 + r]],
                    g_v.at[bp, pl.ds(r * 128, 128)],
                    sem_g[bp],
                ).wait()

        def compute(bp):
            def block(t, c2):
                vals = [
                    [
                        g_v[bp, 4 * t + q, pl.ds(d * 16, 16)]
                        for q in range(RVQ)
                    ]
                    for d in range(DIM // 16)
                ]
                for d in range(DIM // 16):
                    v0, v1, v2, v3 = vals[d]
                    out_v[bp, t, pl.ds(d * 16, 16)] = (v0 + v1) + (v2 + v3)
                return c2

            lax.fori_loop(0, SUB_ROWS * 32, block, 0)

        def p1_idx_copy(tau, slot, sync=False):
            gtau = wid * tiles_pw + tau
            blk = gtau // FULL_TILES
            c = gtau % FULL_TILES
            cp = pltpu.make_async_copy(
                x_hbm.at[pl.ds(blk * 8, 8), pl.ds(c * 128, 128)],
                idx_v.at[slot],
                sem_i[slot],
            )
            if sync:
                cp.start()
                cp.wait()
            return cp

        def p1_out_copy(tau, u, bp):
            gtau = wid * tiles_pw + tau
            blk = gtau // FULL_TILES
            c = gtau % FULL_TILES
            row0 = u * SUB_ROWS
            return [
                pltpu.make_async_copy(
                    out_v.at[bp, pl.ds(r * 32, 32)],
                    out_hbm.at[blk * 8 + row0 + r, pl.ds(c * 32, 32)],
                    sem_o[bp],
                )
                for r in range(SUB_ROWS)
            ]

        _pipeline(p1_idx_copy, p1_out_copy, fire_gathers, wait_gathers,
                  compute, tiles_pw)

        def p2_idx_copy(kk, slot, sync=False):
            cp = pltpu.make_async_copy(
                xt_hbm.at[pl.ds(wid * (t2_tiles * 8) + kk * 8, 8)],
                idx_v.at[slot],
                sem_i[slot],
            )
            if sync:
                cp.start()
                cp.wait()
            return cp

        def p2_out_copy(kk, u, bp):
            u0 = wid * (t2_tiles * 8) + kk * 8 + u * SUB_ROWS
            return [
                pltpu.make_async_copy(
                    out_v.at[bp, pl.ds(r * 32 + bi * 8, 8)],
                    out_hbm.at[(u0 + r) * 4 + bi, pl.ds(FULL_TILES * 32, 8)],
                    sem_o[bp],
                )
                for r in range(SUB_ROWS)
                for bi in range(4)
            ]

        _pipeline(p2_idx_copy, p2_out_copy, fire_gathers, wait_gathers,
                  compute, t2_tiles)

    return k


def kernel(x, emb):
    B, W = x.shape
    L = W // RVQ
    assert W == FULL_TILES * 128 + TAIL_W
    emb_p = jnp.pad(emb, ((0, 0), (0, PDIM - DIM)))
    xt = x[:, FULL_TILES * 128:].reshape(B * TAIL_W // 128, 128)
    return _build(B, L)(x, xt, emb_p)
